# Initial kernel scaffold; baseline (speedup 1.0000x reference)
import jax
import jax.numpy as jnp
from jax.experimental import pallas as pl


def kernel(points, alpha):
    raise NotImplementedError("write your pallas kernel here")


# speedup vs baseline: 7.9917x; 7.9917x over previous
import functools

import jax
import jax.numpy as jnp
from jax import lax
from jax.experimental import pallas as pl
from jax.experimental.pallas import tpu as pltpu, tpu_sc as plsc

_NUM_HASHES = 3


def _sc_row_gather(table, idx):
    N, D = table.shape
    info = plsc.get_sparse_core_info()
    NC, NS = info.num_cores, info.num_subcores
    NW = NC * NS
    assert N % NW == 0
    b_per_w = N // NW
    CH = 32
    assert b_per_w % CH == 0
    n_chunks = b_per_w // CH

    mesh = plsc.VectorSubcoreMesh(core_axis_name="c", subcore_axis_name="s")

    @functools.partial(
        pl.kernel,
        mesh=mesh,
        out_type=jax.ShapeDtypeStruct((N, D), jnp.float32),
        scratch_types=[
            pltpu.VMEM((CH,), jnp.int32),
            pltpu.VMEM((CH, D), jnp.float32),
            pltpu.SemaphoreType.DMA,
        ],
    )
    def gather_kernel(table_hbm, idx_hbm, out_hbm, idx_v, rows_v, sem):
        wid = lax.axis_index("s") * NC + lax.axis_index("c")
        base = wid * b_per_w

        def body(c, _):
            off = base + c * CH
            pltpu.sync_copy(idx_hbm.at[pl.ds(off, CH)], idx_v)
            pltpu.async_copy(table_hbm.at[idx_v], rows_v, sem).wait()
            pltpu.sync_copy(rows_v, out_hbm.at[pl.ds(off, CH)])
            return ()

        lax.fori_loop(0, n_chunks, body, ())

    return gather_kernel(table, idx)


def kernel(points, alpha):
    B, L, D = points.shape
    flat = points.reshape(B * L, D)
    proj = flat @ alpha[0]
    q_h = jnp.transpose(proj)[..., None]
    k_h = q_h
    max_h = jnp.maximum(q_h.max(-1, keepdims=True), k_h.max(-1, keepdims=True))
    min_h = jnp.minimum(q_h.min(-1, keepdims=True), k_h.min(-1, keepdims=True))
    shift = max_h - min_h
    q_shifted = q_h + shift
    scores = q_shifted.sum(-1)
    bucket_matrix = jnp.transpose(scores.reshape(_NUM_HASHES, B, L), (1, 2, 0))
    exponents = (2.0 ** jnp.arange(_NUM_HASHES)).astype(points.dtype)
    bucket_keys = (bucket_matrix * exponents).sum(-1)
    norm = jnp.linalg.norm(points, axis=2)
    norm_idx = jnp.argsort(norm, axis=1)
    bk = jnp.take_along_axis(bucket_keys, norm_idx, axis=1)
    bk_idx = jnp.argsort(bk, axis=1, stable=True)
    indices = jnp.take_along_axis(norm_idx, bk_idx, axis=1)
    indices = jax.lax.stop_gradient(indices)

    flat_idx = (indices + jnp.arange(B, dtype=indices.dtype)[:, None] * L).reshape(-1)
    ordered = _sc_row_gather(flat, flat_idx).reshape(B, L, D)
    return ordered, indices


# speedup vs baseline: 9.6878x; 1.2122x over previous
import functools

import jax
import jax.numpy as jnp
from jax import lax
from jax.experimental import pallas as pl
from jax.experimental.pallas import tpu as pltpu, tpu_sc as plsc

_NUM_HASHES = 3


def _norm_body(x_ref, norm_ref):
    x = x_ref[...]
    norm_ref[...] = jnp.sqrt(jnp.sum(x * x, axis=1, keepdims=True))


def _norm_stage(flat, blk=4096):
    N, D = flat.shape
    return pl.pallas_call(
        _norm_body,
        grid=(N // blk,),
        in_specs=[pl.BlockSpec((blk, D), lambda i: (i, 0))],
        out_specs=pl.BlockSpec((blk, 1), lambda i: (i, 0)),
        out_shape=jax.ShapeDtypeStruct((N, 1), jnp.float32),
    )(flat)


def _roll(x, amt, axis):
    if amt == 0:
        return x
    lo = [slice(None)] * x.ndim
    hi = [slice(None)] * x.ndim
    lo[axis] = slice(amt, None)
    hi[axis] = slice(0, amt)
    return jnp.concatenate([x[tuple(lo)], x[tuple(hi)]], axis=axis)


def _sort_body(keys_ref, norms_ref, idx_out_ref):
    B, R, C = keys_ref.shape
    K = keys_ref[...]
    Nm = norms_ref[...]
    row = lax.broadcasted_iota(jnp.int32, (B, R, C), 1)
    lane = lax.broadcasted_iota(jnp.int32, (B, R, C), 2)
    I = row * C + lane

    n = R * C
    k = 2
    while k <= n:
        j = k // 2
        while j >= 1:
            if j < C:
                lo = (lane & j) == 0
                Kp = jnp.where(lo, _roll(K, j, 2), _roll(K, C - j, 2))
                Np = jnp.where(lo, _roll(Nm, j, 2), _roll(Nm, C - j, 2))
                Ip = jnp.where(lo, _roll(I, j, 2), _roll(I, C - j, 2))
            else:
                m = j // C
                lo = (row & m) == 0
                Kp = jnp.where(lo, _roll(K, m, 1), _roll(K, R - m, 1))
                Np = jnp.where(lo, _roll(Nm, m, 1), _roll(Nm, R - m, 1))
                Ip = jnp.where(lo, _roll(I, m, 1), _roll(I, R - m, 1))
            le = (K < Kp) | ((K == Kp) & ((Nm < Np) | ((Nm == Np) & (I < Ip))))
            if k < C:
                asc = (lane & k) == 0
            elif k < n:
                asc = (row & (k // C)) == 0
            else:
                asc = True
            take_self = (lo == asc) == le
            K = jnp.where(take_self, K, Kp)
            Nm = jnp.where(take_self, Nm, Np)
            I = jnp.where(take_self, I, Ip)
            j //= 2
        k *= 2
    idx_out_ref[...] = I


def _sort_stage(keys, norms):
    B, R, C = keys.shape
    idx = pl.pallas_call(
        _sort_body,
        out_shape=jax.ShapeDtypeStruct((B, R, C), jnp.int32),
    )(keys, norms)
    return idx.reshape(B, R * C)


def _sc_row_gather(table, idx):
    N, D = table.shape
    info = plsc.get_sparse_core_info()
    NC, NS = info.num_cores, info.num_subcores
    NW = NC * NS
    b_per_w = N // NW
    CH = 32
    n_chunks = b_per_w // CH

    mesh = plsc.VectorSubcoreMesh(core_axis_name="c", subcore_axis_name="s")

    @functools.partial(
        pl.kernel,
        mesh=mesh,
        out_type=jax.ShapeDtypeStruct((N, D), jnp.float32),
        scratch_types=[
            pltpu.VMEM((CH,), jnp.int32),
            pltpu.VMEM((CH, D), jnp.float32),
            pltpu.SemaphoreType.DMA,
        ],
    )
    def gather_kernel(table_hbm, idx_hbm, out_hbm, idx_v, rows_v, sem):
        wid = lax.axis_index("s") * NC + lax.axis_index("c")
        base = wid * b_per_w

        def body(c, _):
            off = base + c * CH
            pltpu.sync_copy(idx_hbm.at[pl.ds(off, CH)], idx_v)
            pltpu.async_copy(table_hbm.at[idx_v], rows_v, sem).wait()
            pltpu.sync_copy(rows_v, out_hbm.at[pl.ds(off, CH)])
            return ()

        lax.fori_loop(0, n_chunks, body, ())

    return gather_kernel(table, idx)


def kernel(points, alpha):
    B, L, D = points.shape
    R, C = L // 128, 128
    flat = points.reshape(B * L, D)

    proj = flat @ alpha[0]
    q_h = jnp.transpose(proj)[..., None]
    k_h = q_h
    max_h = jnp.maximum(q_h.max(-1, keepdims=True), k_h.max(-1, keepdims=True))
    min_h = jnp.minimum(q_h.min(-1, keepdims=True), k_h.min(-1, keepdims=True))
    shift = max_h - min_h
    scores = (q_h + shift).sum(-1)
    bucket_matrix = jnp.transpose(scores.reshape(_NUM_HASHES, B, L), (1, 2, 0))
    exponents = (2.0 ** jnp.arange(_NUM_HASHES)).astype(points.dtype)
    bucket_keys = (bucket_matrix * exponents).sum(-1)

    norms = _norm_stage(flat).reshape(B, R, C)
    indices = _sort_stage(bucket_keys.reshape(B, R, C), norms)

    flat_idx = (indices + jnp.arange(B, dtype=indices.dtype)[:, None] * L).reshape(-1)
    ordered = _sc_row_gather(flat, flat_idx).reshape(B, L, D)
    return ordered, indices


# speedup vs baseline: 11.1750x; 1.1535x over previous
import functools

import jax
import jax.numpy as jnp
from jax import lax
from jax.experimental import pallas as pl
from jax.experimental.pallas import tpu as pltpu, tpu_sc as plsc

_NUM_HASHES = 3


def _norm_body(x_ref, norm_ref):
    x = x_ref[...]
    norm_ref[...] = jnp.sqrt(jnp.sum(x * x, axis=1, keepdims=True))


def _norm_stage(flat, blk=4096):
    N, D = flat.shape
    return pl.pallas_call(
        _norm_body,
        grid=(N // blk,),
        in_specs=[pl.BlockSpec((blk, D), lambda i: (i, 0))],
        out_specs=pl.BlockSpec((blk, 1), lambda i: (i, 0)),
        out_shape=jax.ShapeDtypeStruct((N, 1), jnp.float32),
    )(flat)


def _roll(x, amt, axis):
    if amt == 0:
        return x
    lo = [slice(None)] * x.ndim
    hi = [slice(None)] * x.ndim
    lo[axis] = slice(amt, None)
    hi[axis] = slice(0, amt)
    return jnp.concatenate([x[tuple(lo)], x[tuple(hi)]], axis=axis)


def _sort_body(keys_ref, norms_ref, idx_out_ref, flat_out_ref):
    B, R, C = keys_ref.shape
    K = keys_ref[...]
    Nm = norms_ref[...]
    row = lax.broadcasted_iota(jnp.int32, (B, R, C), 1)
    lane = lax.broadcasted_iota(jnp.int32, (B, R, C), 2)
    I = row * C + lane

    n = R * C
    k = 2
    while k <= n:
        j = k // 2
        while j >= 1:
            if j < C:
                lo = (lane & j) == 0
                Kp = jnp.where(lo, _roll(K, j, 2), _roll(K, C - j, 2))
                Np = jnp.where(lo, _roll(Nm, j, 2), _roll(Nm, C - j, 2))
                Ip = jnp.where(lo, _roll(I, j, 2), _roll(I, C - j, 2))
            else:
                m = j // C
                lo = (row & m) == 0
                Kp = jnp.where(lo, _roll(K, m, 1), _roll(K, R - m, 1))
                Np = jnp.where(lo, _roll(Nm, m, 1), _roll(Nm, R - m, 1))
                Ip = jnp.where(lo, _roll(I, m, 1), _roll(I, R - m, 1))
            le = (K < Kp) | ((K == Kp) & ((Nm < Np) | ((Nm == Np) & (I < Ip))))
            if k < C:
                asc = (lane & k) == 0
            elif k < n:
                asc = (row & (k // C)) == 0
            else:
                asc = True
            take_self = (lo == asc) == le
            K = jnp.where(take_self, K, Kp)
            Nm = jnp.where(take_self, Nm, Np)
            I = jnp.where(take_self, I, Ip)
            j //= 2
        k *= 2
    idx_out_ref[...] = I
    batch = lax.broadcasted_iota(jnp.int32, (B, R, C), 0)
    flat_out_ref[...] = I + batch * (R * C)


def _sort_stage(keys, norms):
    B, R, C = keys.shape
    idx, flat_idx = pl.pallas_call(
        _sort_body,
        out_shape=[
            jax.ShapeDtypeStruct((B, R, C), jnp.int32),
            jax.ShapeDtypeStruct((B, R, C), jnp.int32),
        ],
    )(keys, norms)
    return idx.reshape(B, R * C), flat_idx.reshape(B * R * C)


def _sc_row_gather(table, idx):
    N, D = table.shape
    info = plsc.get_sparse_core_info()
    NC, NS = info.num_cores, info.num_subcores
    NW = NC * NS
    b_per_w = N // NW
    CH = 32
    n_chunks = b_per_w // CH

    n_pairs = n_chunks // 2
    mesh = plsc.VectorSubcoreMesh(core_axis_name="c", subcore_axis_name="s")

    @functools.partial(
        pl.kernel,
        mesh=mesh,
        out_type=jax.ShapeDtypeStruct((N, D), jnp.float32),
        scratch_types=[
            pltpu.VMEM((CH,), jnp.int32),
            pltpu.VMEM((CH,), jnp.int32),
            pltpu.VMEM((CH, D), jnp.float32),
            pltpu.VMEM((CH, D), jnp.float32),
            pltpu.SemaphoreType.DMA,
            pltpu.SemaphoreType.DMA,
        ],
    )
    def gather_kernel(table_hbm, idx_hbm, out_hbm,
                      idx0, idx1, rows0, rows1, sem0, sem1):
        wid = lax.axis_index("s") * NC + lax.axis_index("c")
        base = wid * b_per_w

        pltpu.sync_copy(idx_hbm.at[pl.ds(base, CH)], idx0)
        pltpu.async_copy(table_hbm.at[idx0], rows0, sem0)

        def pair(p, _):
            c0 = base + 2 * p * CH
            pltpu.sync_copy(idx_hbm.at[pl.ds(c0 + CH, CH)], idx1)
            pltpu.async_copy(table_hbm.at[idx1], rows1, sem1)
            pltpu.make_async_copy(table_hbm.at[idx0], rows0, sem0).wait()
            pltpu.sync_copy(rows0, out_hbm.at[pl.ds(c0, CH)])

            @pl.when(p + 1 < n_pairs)
            def _():
                pltpu.sync_copy(idx_hbm.at[pl.ds(c0 + 2 * CH, CH)], idx0)
                pltpu.async_copy(table_hbm.at[idx0], rows0, sem0)

            pltpu.make_async_copy(table_hbm.at[idx1], rows1, sem1).wait()
            pltpu.sync_copy(rows1, out_hbm.at[pl.ds(c0 + CH, CH)])
            return ()

        lax.fori_loop(0, n_pairs, pair, ())

    return gather_kernel(table, idx)


def kernel(points, alpha):
    B, L, D = points.shape
    R, C = L // 128, 128
    flat = points.reshape(B * L, D)

    proj = flat @ alpha[0]
    q_h = jnp.transpose(proj)[..., None]
    k_h = q_h
    max_h = jnp.maximum(q_h.max(-1, keepdims=True), k_h.max(-1, keepdims=True))
    min_h = jnp.minimum(q_h.min(-1, keepdims=True), k_h.min(-1, keepdims=True))
    shift = max_h - min_h
    scores = (q_h + shift).sum(-1)
    bucket_matrix = jnp.transpose(scores.reshape(_NUM_HASHES, B, L), (1, 2, 0))
    exponents = (2.0 ** jnp.arange(_NUM_HASHES)).astype(points.dtype)
    bucket_keys = (bucket_matrix * exponents).sum(-1)

    norms = _norm_stage(flat).reshape(B, R, C)
    indices, flat_idx = _sort_stage(bucket_keys.reshape(B, R, C), norms)

    ordered = _sc_row_gather(flat, flat_idx).reshape(B, L, D)
    return ordered, indices


# speedup vs baseline: 11.2082x; 1.0030x over previous
import functools

import jax
import jax.numpy as jnp
from jax import lax
from jax.experimental import pallas as pl
from jax.experimental.pallas import tpu as pltpu, tpu_sc as plsc

_NUM_HASHES = 3


def _norm_body(x_ref, norm_ref):
    x = x_ref[...]
    norm_ref[...] = jnp.sqrt(jnp.sum(x * x, axis=1, keepdims=True))


def _norm_stage(flat, blk=2048):
    N, D = flat.shape
    return pl.pallas_call(
        _norm_body,
        grid=(N // blk,),
        in_specs=[pl.BlockSpec((blk, D), lambda i: (i, 0))],
        out_specs=pl.BlockSpec((blk, 1), lambda i: (i, 0)),
        out_shape=jax.ShapeDtypeStruct((N, 1), jnp.float32),
    )(flat)


def _roll(x, amt, axis):
    if amt == 0:
        return x
    lo = [slice(None)] * x.ndim
    hi = [slice(None)] * x.ndim
    lo[axis] = slice(amt, None)
    hi[axis] = slice(0, amt)
    return jnp.concatenate([x[tuple(lo)], x[tuple(hi)]], axis=axis)


def _sort_body(keys_ref, norms_ref, idx_out_ref, flat_out_ref):
    B, R, C = keys_ref.shape
    K = keys_ref[...]
    Nm = norms_ref[...]
    row = lax.broadcasted_iota(jnp.int32, (B, R, C), 1)
    lane = lax.broadcasted_iota(jnp.int32, (B, R, C), 2)
    I = row * C + lane

    n = R * C
    k = 2
    while k <= n:
        j = k // 2
        while j >= 1:
            if j < C:
                lo = (lane & j) == 0
                Kp = jnp.where(lo, _roll(K, j, 2), _roll(K, C - j, 2))
                Np = jnp.where(lo, _roll(Nm, j, 2), _roll(Nm, C - j, 2))
                Ip = jnp.where(lo, _roll(I, j, 2), _roll(I, C - j, 2))
            else:
                m = j // C
                lo = (row & m) == 0
                Kp = jnp.where(lo, _roll(K, m, 1), _roll(K, R - m, 1))
                Np = jnp.where(lo, _roll(Nm, m, 1), _roll(Nm, R - m, 1))
                Ip = jnp.where(lo, _roll(I, m, 1), _roll(I, R - m, 1))
            le = (K < Kp) | ((K == Kp) & ((Nm < Np) | ((Nm == Np) & (I < Ip))))
            if k < C:
                asc = (lane & k) == 0
            elif k < n:
                asc = (row & (k // C)) == 0
            else:
                asc = True
            take_self = (lo == asc) == le
            K = jnp.where(take_self, K, Kp)
            Nm = jnp.where(take_self, Nm, Np)
            I = jnp.where(take_self, I, Ip)
            j //= 2
        k *= 2
    idx_out_ref[...] = I
    batch = lax.broadcasted_iota(jnp.int32, (B, R, C), 0)
    flat_out_ref[...] = I + batch * (R * C)


def _sort_stage(keys, norms):
    B, R, C = keys.shape
    idx, flat_idx = pl.pallas_call(
        _sort_body,
        out_shape=[
            jax.ShapeDtypeStruct((B, R, C), jnp.int32),
            jax.ShapeDtypeStruct((B, R, C), jnp.int32),
        ],
    )(keys, norms)
    return idx.reshape(B, R * C), flat_idx.reshape(B * R * C)


def _sc_row_gather(table, idx):
    N, D = table.shape
    info = plsc.get_sparse_core_info()
    NC, NS = info.num_cores, info.num_subcores
    NW = NC * NS
    b_per_w = N // NW
    CH = 32
    n_chunks = b_per_w // CH

    n_pairs = n_chunks // 2
    mesh = plsc.VectorSubcoreMesh(core_axis_name="c", subcore_axis_name="s")

    @functools.partial(
        pl.kernel,
        mesh=mesh,
        out_type=jax.ShapeDtypeStruct((N, D), jnp.float32),
        scratch_types=[
            pltpu.VMEM((CH,), jnp.int32),
            pltpu.VMEM((CH,), jnp.int32),
            pltpu.VMEM((CH, D), jnp.float32),
            pltpu.VMEM((CH, D), jnp.float32),
            pltpu.SemaphoreType.DMA,
            pltpu.SemaphoreType.DMA,
        ],
    )
    def gather_kernel(table_hbm, idx_hbm, out_hbm,
                      idx0, idx1, rows0, rows1, sem0, sem1):
        wid = lax.axis_index("s") * NC + lax.axis_index("c")
        base = wid * b_per_w

        pltpu.sync_copy(idx_hbm.at[pl.ds(base, CH)], idx0)
        pltpu.async_copy(table_hbm.at[idx0], rows0, sem0)

        def pair(p, _):
            c0 = base + 2 * p * CH
            pltpu.sync_copy(idx_hbm.at[pl.ds(c0 + CH, CH)], idx1)
            pltpu.async_copy(table_hbm.at[idx1], rows1, sem1)
            pltpu.make_async_copy(table_hbm.at[idx0], rows0, sem0).wait()
            pltpu.sync_copy(rows0, out_hbm.at[pl.ds(c0, CH)])

            @pl.when(p + 1 < n_pairs)
            def _():
                pltpu.sync_copy(idx_hbm.at[pl.ds(c0 + 2 * CH, CH)], idx0)
                pltpu.async_copy(table_hbm.at[idx0], rows0, sem0)

            pltpu.make_async_copy(table_hbm.at[idx1], rows1, sem1).wait()
            pltpu.sync_copy(rows1, out_hbm.at[pl.ds(c0 + CH, CH)])
            return ()

        lax.fori_loop(0, n_pairs, pair, ())

    return gather_kernel(table, idx)


def kernel(points, alpha):
    B, L, D = points.shape
    R, C = L // 128, 128
    flat = points.reshape(B * L, D)

    proj = flat @ alpha[0]
    q_h = jnp.transpose(proj)[..., None]
    k_h = q_h
    max_h = jnp.maximum(q_h.max(-1, keepdims=True), k_h.max(-1, keepdims=True))
    min_h = jnp.minimum(q_h.min(-1, keepdims=True), k_h.min(-1, keepdims=True))
    shift = max_h - min_h
    scores = (q_h + shift).sum(-1)
    bucket_matrix = jnp.transpose(scores.reshape(_NUM_HASHES, B, L), (1, 2, 0))
    exponents = (2.0 ** jnp.arange(_NUM_HASHES)).astype(points.dtype)
    bucket_keys = (bucket_matrix * exponents).sum(-1)

    norms = _norm_stage(flat).reshape(B, R, C)
    indices, flat_idx = _sort_stage(bucket_keys.reshape(B, R, C), norms)

    ordered = _sc_row_gather(flat, flat_idx).reshape(B, L, D)
    return ordered, indices
